# Initial kernel scaffold; baseline (speedup 1.0000x reference)
import functools

import jax
import jax.numpy as jnp
from jax import lax
from jax.experimental import pallas as pl
from jax.experimental.pallas import tpu as pltpu
from jax.experimental.pallas import tpu_sc as plsc

N = 10000
E = 160000
H = 128
NUM_CLASSES = 4

NC = 2
NS = 16
NW = NC * NS
NPW = 320
N_PAD = NW * NPW
B = 128
CAP = E + NW * B
NEG = jnp.float32(-1e9)


def _agg_body(h_hbm, src_hbm, dl_hbm, ew_hbm, stc_hbm, nch_hbm, agg_hbm,
              agg_v, rows_v, idx_v, dl_v, ew_v, m1_v, m2_v, sem):
    cid = lax.axis_index("c")
    sid = lax.axis_index("s")
    wid = sid * NC + cid

    pltpu.sync_copy(stc_hbm.at[wid], m1_v)
    pltpu.sync_copy(nch_hbm.at[wid], m2_v)
    startc = jnp.max(m1_v[...])
    nch = jnp.max(m2_v[...])

    def init_body(i, _):
        agg_v[pl.ds(i * 16, 16)] = jnp.full((16,), NEG, jnp.float32)
        return _
    lax.fori_loop(0, (NPW + 1) * H // 16, init_body, None)

    lane = lax.iota(jnp.int32, 16)

    def chunk_body(c, _):
        off = (startc + c) * B
        pltpu.sync_copy(src_hbm.at[pl.ds(off, B)], idx_v)
        cp = pltpu.async_copy(h_hbm.at[idx_v], rows_v, sem)
        pltpu.sync_copy(dl_hbm.at[pl.ds(off, B)], dl_v)
        pltpu.sync_copy(ew_hbm.at[pl.ds(off, B)], ew_v)
        cp.wait()

        def group_body(g, _):
            dlv = dl_v[pl.ds(g * 16, 16)]
            ewv = ew_v[pl.ds(g * 16, 16)]
            for l in range(16):
                d = jnp.max(jnp.where(lane == l, dlv, 0))
                w = jnp.max(jnp.where(lane == l, ewv, jnp.float32("-inf")))
                rowbase = d * H
                erow = g * 16 + l
                for f in range(H // 16):
                    r = rows_v[erow, pl.ds(f * 16, 16)]
                    a = agg_v[pl.ds(rowbase + f * 16, 16)]
                    agg_v[pl.ds(rowbase + f * 16, 16)] = jnp.maximum(a, r * w)
            return _
        lax.fori_loop(0, B // 16, group_body, None)
        return _

    lax.fori_loop(0, nch, chunk_body, None)
    pltpu.sync_copy(agg_v.at[pl.ds(0, NPW * H)],
                    agg_hbm.at[pl.ds(wid * NPW * H, NPW * H)])


_agg_call = pl.kernel(
    _agg_body,
    out_type=jax.ShapeDtypeStruct((N_PAD * H,), jnp.float32),
    mesh=plsc.VectorSubcoreMesh(core_axis_name="c", subcore_axis_name="s",
                                num_cores=NC, num_subcores=NS),
    scratch_types=[
        pltpu.VMEM(((NPW + 1) * H,), jnp.float32),
        pltpu.VMEM((B, H), jnp.float32),
        pltpu.VMEM((B,), jnp.int32),
        pltpu.VMEM((B,), jnp.int32),
        pltpu.VMEM((B,), jnp.float32),
        pltpu.VMEM((16,), jnp.int32),
        pltpu.VMEM((16,), jnp.int32),
        pltpu.SemaphoreType.DMA,
    ],
)


def _mlp_body(scale_ref, h_ref, agg_ref, w1_ref, b1_ref, w2_ref, b2_ref,
              out_ref):
    s = scale_ref[0, 0]
    a = agg_ref[...]
    a = jnp.where(a == NEG, 0.0, a)
    z = s * h_ref[...] + a
    z = jnp.dot(z, w1_ref[...], preferred_element_type=jnp.float32) + b1_ref[...]
    z = jnp.where(z >= 0, z, 0.01 * z)
    z = jnp.dot(z, w2_ref[...], preferred_element_type=jnp.float32) + b2_ref[...]
    out_ref[...] = jnp.where(z >= 0, z, 0.01 * z)


def _mlp_fc_body(scale_ref, h_ref, agg_ref, w1_ref, b1_ref, w2_ref, b2_ref,
                 w3_ref, b3_ref, out_ref):
    s = scale_ref[0, 0]
    a = agg_ref[...]
    a = jnp.where(a == NEG, 0.0, a)
    z = s * h_ref[...] + a
    z = jnp.dot(z, w1_ref[...], preferred_element_type=jnp.float32) + b1_ref[...]
    z = jnp.where(z >= 0, z, 0.01 * z)
    z = jnp.dot(z, w2_ref[...], preferred_element_type=jnp.float32) + b2_ref[...]
    z = jnp.where(z >= 0, z, 0.01 * z)
    out_ref[...] = jnp.dot(z, w3_ref[...], preferred_element_type=jnp.float32) + b3_ref[...]


BLK = 1024


def _mlp_call(scale, h, agg, w1, b1, w2, b2, w3=None, b3=None):
    full = lambda shp: pl.BlockSpec(shp, lambda i: (0, 0))
    specs = [
        pl.BlockSpec(memory_space=pltpu.SMEM),
        pl.BlockSpec((BLK, H), lambda i: (i, 0)),
        pl.BlockSpec((BLK, H), lambda i: (i, 0)),
        full((H, H)), full((1, H)),
        full((H, H)), full((1, H)),
    ]
    args = [scale, h, agg, w1, b1, w2, b2]
    body = _mlp_body
    if w3 is not None:
        specs += [full((H, H)), full((1, H))]
        args += [w3, b3]
        body = _mlp_fc_body
    return pl.pallas_call(
        body,
        grid=(N_PAD // BLK,),
        in_specs=specs,
        out_specs=pl.BlockSpec((BLK, H), lambda i: (i, 0)),
        out_shape=jax.ShapeDtypeStruct((N_PAD, H), jnp.float32),
    )(*args)


def kernel(x, edge_index, edge_attr, params):
    src = edge_index[0].astype(jnp.int32)
    dst = edge_index[1].astype(jnp.int32)
    ew = edge_attr[:, 0]

    order = jnp.argsort(dst)
    src_s = src[order]
    dst_s = dst[order]
    ew_s = ew[order]
    bucket_s = dst_s // NPW
    counts = jnp.bincount(bucket_s, length=NW).astype(jnp.int32)
    cp = ((counts + B - 1) // B) * B
    cpc = jnp.cumsum(cp)
    cstart = cpc - cp
    ecstart = jnp.cumsum(counts) - counts

    p = jnp.arange(CAP, dtype=jnp.int32)
    wp = jnp.minimum(jnp.searchsorted(cpc, p, side="right"), NW - 1)
    r = p - cstart[wp]
    valid = r < counts[wp]
    gidx = jnp.where(valid, ecstart[wp] + r, 0)
    src_p = jnp.where(valid, src_s[gidx], 0)
    ew_p = jnp.where(valid, ew_s[gidx], 0.0)
    dl_p = jnp.where(valid, dst_s[gidx] - wp * NPW, NPW).astype(jnp.int32)
    nch_rep = jnp.broadcast_to((cp // B)[:, None], (NW, 16)).astype(jnp.int32)
    stc_rep = jnp.broadcast_to((cstart // B)[:, None], (NW, 16)).astype(jnp.int32)

    h = jnp.zeros((N_PAD, H), jnp.float32).at[:N, :x.shape[1]].set(x)

    layers = params["layers"]
    nl = len(layers)
    for i, lp in enumerate(layers):
        w1 = lp["W1"]
        if w1.shape[0] < H:
            w1 = jnp.zeros((H, H), jnp.float32).at[:w1.shape[0]].set(w1)
        scale = (1.0 + lp["eps"]).reshape(1, 1)
        agg = _agg_call(h, src_p, dl_p, ew_p, stc_rep, nch_rep)
        agg = agg.reshape(N_PAD, H)
        if i + 1 < nl:
            h = _mlp_call(scale, h, agg, w1, lp["b1"].reshape(1, H),
                          lp["W2"], lp["b2"].reshape(1, H))
        else:
            w3 = jnp.zeros((H, H), jnp.float32).at[:, :NUM_CLASSES].set(params["fc_W"])
            b3 = jnp.zeros((1, H), jnp.float32).at[0, :NUM_CLASSES].set(params["fc_b"])
            h = _mlp_call(scale, h, agg, w1, lp["b1"].reshape(1, H),
                          lp["W2"], lp["b2"].reshape(1, H), w3, b3)
    return h[:N, :NUM_CLASSES]


# speedup vs baseline: 1.7096x; 1.7096x over previous
import functools

import jax
import jax.numpy as jnp
from jax import lax
from jax.experimental import pallas as pl
from jax.experimental.pallas import tpu as pltpu
from jax.experimental.pallas import tpu_sc as plsc

N = 10000
E = 160000
H = 128
NUM_CLASSES = 4

NC = 2
NS = 16
NW = NC * NS
NPW = 320
N_PAD = NW * NPW
B = 128
CAP = E + NW * B
NEG = -1e9


def _agg_body(h_hbm, src_hbm, dl_hbm, ew_hbm, meta_hbm, agg_hbm,
              agg_v, rows_v, idx_v, dl_v, ew_v, m_v, sem):
    cid = lax.axis_index("c")
    sid = lax.axis_index("s")
    wid = sid * NC + cid

    pltpu.sync_copy(meta_hbm.at[wid], m_v)
    startc = m_v[pl.ds(0, 16)][0]
    nch = m_v[pl.ds(1, 16)][0]

    def init_body(i, _):
        agg_v[pl.ds(i * 16, 16)] = jnp.full((16,), NEG, jnp.float32)
        return _
    lax.fori_loop(0, (NPW + 1) * H // 16, init_body, None)

    def chunk_body(c, _):
        off = (startc + c) * B
        pltpu.sync_copy(src_hbm.at[pl.ds(off, B)], idx_v)
        cp = pltpu.async_copy(h_hbm.at[idx_v], rows_v, sem)
        pltpu.sync_copy(dl_hbm.at[pl.ds(off, B)], dl_v.at[pl.ds(0, B)])
        pltpu.sync_copy(ew_hbm.at[pl.ds(off, B)], ew_v.at[pl.ds(0, B)])
        cp.wait()

        def edge_body(e, _):
            d = dl_v[pl.ds(e, 16)][0]
            w = ew_v[pl.ds(e, 16)][0]
            rowbase = d * H
            for f in range(H // 16):
                r = rows_v[e, pl.ds(f * 16, 16)]
                a = agg_v[pl.ds(rowbase + f * 16, 16)]
                agg_v[pl.ds(rowbase + f * 16, 16)] = jnp.maximum(a, r * w)
            return _
        lax.fori_loop(0, B, edge_body, None)
        return _

    lax.fori_loop(0, nch, chunk_body, None)
    pltpu.sync_copy(agg_v.at[pl.ds(0, NPW * H)],
                    agg_hbm.at[pl.ds(wid * NPW * H, NPW * H)])


_agg_call = pl.kernel(
    _agg_body,
    out_type=jax.ShapeDtypeStruct((N_PAD * H,), jnp.float32),
    mesh=plsc.VectorSubcoreMesh(core_axis_name="c", subcore_axis_name="s",
                                num_cores=NC, num_subcores=NS),
    scratch_types=[
        pltpu.VMEM(((NPW + 1) * H,), jnp.float32),
        pltpu.VMEM((B, H), jnp.float32),
        pltpu.VMEM((B,), jnp.int32),
        pltpu.VMEM((B + 16,), jnp.int32),
        pltpu.VMEM((B + 16,), jnp.float32),
        pltpu.VMEM((32,), jnp.int32),
        pltpu.SemaphoreType.DMA,
    ],
)


def _mlp_body(scale_ref, h_ref, agg_ref, w1_ref, b1_ref, w2_ref, b2_ref,
              out_ref):
    s = scale_ref[0, 0]
    a = agg_ref[...]
    a = jnp.where(a == NEG, 0.0, a)
    z = s * h_ref[...] + a
    z = jnp.dot(z, w1_ref[...], preferred_element_type=jnp.float32) + b1_ref[...]
    z = jnp.where(z >= 0, z, 0.01 * z)
    z = jnp.dot(z, w2_ref[...], preferred_element_type=jnp.float32) + b2_ref[...]
    out_ref[...] = jnp.where(z >= 0, z, 0.01 * z)


def _mlp_fc_body(scale_ref, h_ref, agg_ref, w1_ref, b1_ref, w2_ref, b2_ref,
                 w3_ref, b3_ref, out_ref):
    s = scale_ref[0, 0]
    a = agg_ref[...]
    a = jnp.where(a == NEG, 0.0, a)
    z = s * h_ref[...] + a
    z = jnp.dot(z, w1_ref[...], preferred_element_type=jnp.float32) + b1_ref[...]
    z = jnp.where(z >= 0, z, 0.01 * z)
    z = jnp.dot(z, w2_ref[...], preferred_element_type=jnp.float32) + b2_ref[...]
    z = jnp.where(z >= 0, z, 0.01 * z)
    out_ref[...] = jnp.dot(z, w3_ref[...], preferred_element_type=jnp.float32) + b3_ref[...]


BLK = 1024


def _mlp_call(scale, h, agg, w1, b1, w2, b2, w3=None, b3=None):
    full = lambda shp: pl.BlockSpec(shp, lambda i: (0, 0))
    specs = [
        pl.BlockSpec(memory_space=pltpu.SMEM),
        pl.BlockSpec((BLK, H), lambda i: (i, 0)),
        pl.BlockSpec((BLK, H), lambda i: (i, 0)),
        full((H, H)), full((1, H)),
        full((H, H)), full((1, H)),
    ]
    args = [scale, h, agg, w1, b1, w2, b2]
    body = _mlp_body
    if w3 is not None:
        specs += [full((H, H)), full((1, H))]
        args += [w3, b3]
        body = _mlp_fc_body
    return pl.pallas_call(
        body,
        grid=(N_PAD // BLK,),
        in_specs=specs,
        out_specs=pl.BlockSpec((BLK, H), lambda i: (i, 0)),
        out_shape=jax.ShapeDtypeStruct((N_PAD, H), jnp.float32),
    )(*args)


def kernel(x, edge_index, edge_attr, params):
    src = edge_index[0].astype(jnp.int32)
    dst = edge_index[1].astype(jnp.int32)
    ew = edge_attr[:, 0]

    order = jnp.argsort(dst)
    src_s = src[order]
    dst_s = dst[order]
    ew_s = ew[order]
    bucket_s = dst_s // NPW
    counts = jnp.bincount(bucket_s, length=NW).astype(jnp.int32)
    cp = ((counts + B - 1) // B) * B
    cpc = jnp.cumsum(cp)
    cstart = cpc - cp
    ecstart = jnp.cumsum(counts) - counts

    p = jnp.arange(CAP, dtype=jnp.int32)
    wp = jnp.minimum(jnp.searchsorted(cpc, p, side="right"), NW - 1)
    r = p - cstart[wp]
    valid = r < counts[wp]
    gidx = jnp.where(valid, ecstart[wp] + r, 0)
    src_p = jnp.where(valid, src_s[gidx], 0)
    ew_p = jnp.where(valid, ew_s[gidx], 0.0)
    dl_p = jnp.where(valid, dst_s[gidx] - wp * NPW, NPW).astype(jnp.int32)
    meta = jnp.zeros((NW, 32), jnp.int32)
    meta = meta.at[:, 0].set(cstart // B).at[:, 1].set(cp // B)

    h = jnp.zeros((N_PAD, H), jnp.float32).at[:N, :x.shape[1]].set(x)

    layers = params["layers"]
    nl = len(layers)
    for i, lp in enumerate(layers):
        w1 = lp["W1"]
        if w1.shape[0] < H:
            w1 = jnp.zeros((H, H), jnp.float32).at[:w1.shape[0]].set(w1)
        scale = (1.0 + lp["eps"]).reshape(1, 1)
        agg = _agg_call(h, src_p, dl_p, ew_p, meta)
        agg = agg.reshape(N_PAD, H)
        if i + 1 < nl:
            h = _mlp_call(scale, h, agg, w1, lp["b1"].reshape(1, H),
                          lp["W2"], lp["b2"].reshape(1, H))
        else:
            w3 = jnp.zeros((H, H), jnp.float32).at[:, :NUM_CLASSES].set(params["fc_W"])
            b3 = jnp.zeros((1, H), jnp.float32).at[0, :NUM_CLASSES].set(params["fc_b"])
            h = _mlp_call(scale, h, agg, w1, lp["b1"].reshape(1, H),
                          lp["W2"], lp["b2"].reshape(1, H), w3, b3)
    return h[:N, :NUM_CLASSES]
